# baseline (speedup 1.0000x reference)
import functools
import math

import numpy as np
import jax
import jax.numpy as jnp
from jax.experimental import pallas as pl

SEQ_LEN = 512
PRED_LEN = 96
D_MODEL = 256
N_HEADS = 8
DH = D_MODEL // N_HEADS
E_LAYERS = 2
D_FF = 512
ENC_IN = 7
B = 8
PATCHES = [16, 8, 4]
STRIDE = 8
PAD = 8
NE = 3
NROWS = B * ENC_IN

F32 = jnp.float32
BF16 = jnp.bfloat16


def _pn_of(patch):
    return (SEQ_LEN + PAD - patch) // STRIDE + 1


def _rup(v, m):
    return ((v + m - 1) // m) * m


def _sinusoid_pe(max_len, d):
    pe = np.zeros((max_len, d), dtype=np.float32)
    pos = np.arange(max_len, dtype=np.float32)[:, None]
    div = np.exp(np.arange(0, d, 2, dtype=np.float32) * (-np.log(10000.0) / d))
    pe[:, 0::2] = np.sin(pos * div)
    pe[:, 1::2] = np.cos(pos * div)
    return pe


_PE_NP = _sinusoid_pe(128, D_MODEL)

_KF = SEQ_LEN // 2 + 1
_KP = 384
_n = np.arange(SEQ_LEN, dtype=np.float64)[:, None]
_k = np.arange(_KF, dtype=np.float64)[None, :]
_ang = 2.0 * np.pi * _n * _k / SEQ_LEN
_COS = np.zeros((SEQ_LEN, _KP), dtype=np.float32)
_SIN = np.zeros((SEQ_LEN, _KP), dtype=np.float32)
_COS[:, :_KF] = np.cos(_ang).astype(np.float32)
_SIN[:, :_KF] = np.sin(_ang).astype(np.float32)
_CS_NP = np.concatenate([_COS, _SIN], axis=1)

_M_NP = np.zeros((B, NROWS), dtype=np.float32)
for _b in range(B):
    _M_NP[_b, _b * ENC_IN:(_b + 1) * ENC_IN] = 1.0 / ENC_IN

_R_NP = np.zeros((NROWS, B), dtype=np.float32)
for _i in range(NROWS):
    _R_NP[_i, _i // ENC_IN] = 1.0


def _ln(x, g, b):
    m = jnp.mean(x, axis=-1, keepdims=True)
    v = jnp.mean((x - m) ** 2, axis=-1, keepdims=True)
    return (x - m) / jnp.sqrt(v + 1e-5) * g + b


def _gelu(x):
    return 0.5 * x * (1.0 + jax.lax.erf(x / np.float32(math.sqrt(2.0))))


def _gate_body(x_ref, cs_ref, wg_ref, m_ref, gates_ref, aux_ref):
    ri = jnp.dot(x_ref[...], cs_ref[...], preferred_element_type=F32)
    re = ri[:, :_KP]
    im = ri[:, _KP:]
    amp = jnp.sqrt(re * re + im * im)
    l56 = jnp.dot(amp, wg_ref[...], preferred_element_type=F32)
    lg = jnp.dot(m_ref[...], l56, preferred_element_type=F32)
    l0 = lg[:, 0:1]
    l1 = lg[:, 1:2]
    l2 = lg[:, 2:3]
    c01 = l0 >= l1
    c02 = l0 >= l2
    c12 = l1 >= l2
    i0_is0 = c01 & c02
    i0_is1 = (~i0_is0) & c12
    i0_is2 = (~i0_is0) & (~c12)
    v0 = jnp.where(i0_is0, l0, jnp.where(i0_is1, l1, l2))
    i1_is0 = (i0_is1 & c02) | (i0_is2 & c01)
    i1_is1 = (i0_is0 & c12) | (i0_is2 & (~c01))
    i1_is2 = (i0_is0 & (~c12)) | (i0_is1 & (~c02))
    v1 = jnp.where(i1_is0, l0, jnp.where(i1_is1, l1, l2))
    e1w = jnp.exp(v1 - v0)
    denom = 1.0 + e1w
    gf = 1.0 / denom
    gs = e1w / denom
    zero = jnp.zeros_like(gf)
    g_cols = []
    for e, (a, bb) in enumerate(((i0_is0, i1_is0), (i0_is1, i1_is1),
                                 (i0_is2, i1_is2))):
        g_cols.append(jnp.where(a, gf, zero) + jnp.where(bb, gs, zero))
    gates_ref[...] = jnp.zeros((B, 128), F32)
    imp = []
    load = []
    for e in range(NE):
        gates_ref[:, e:e + 1] = g_cols[e]
        imp.append(jnp.sum(g_cols[e]))
        load.append(jnp.sum((g_cols[e] > 0).astype(F32)))

    def cv3(a, bb, c):
        m = (a + bb + c) / 3.0
        var = ((a - m) ** 2 + (bb - m) ** 2 + (c - m) ** 2) / 2.0
        return var / (m * m + 1e-10)

    aux = (cv3(*imp) + cv3(*load)) * 0.01
    aux_ref[...] = jnp.broadcast_to(aux, (1, 128)).astype(F32)


def _gate_call(xs, cs, wg, m):
    return pl.pallas_call(
        _gate_body,
        grid=(1,),
        in_specs=[
            pl.BlockSpec(xs.shape, lambda i: (0, 0)),
            pl.BlockSpec(cs.shape, lambda i: (0, 0)),
            pl.BlockSpec(wg.shape, lambda i: (0, 0)),
            pl.BlockSpec(m.shape, lambda i: (0, 0)),
        ],
        out_specs=[
            pl.BlockSpec((B, 128), lambda i: (0, 0)),
            pl.BlockSpec((1, 128), lambda i: (0, 0)),
        ],
        out_shape=[
            jax.ShapeDtypeStruct((B, 128), F32),
            jax.ShapeDtypeStruct((1, 128), F32),
        ],
    )(xs, cs, wg, m)


def _enc_stack(x, Lp, L_real, RB, lrefs, ng_ref, nb_ref):
    i0 = jax.lax.broadcasted_iota(jnp.int32, (N_HEADS * Lp, D_MODEL), 0)
    i1 = jax.lax.broadcasted_iota(jnp.int32, (N_HEADS * Lp, D_MODEL), 1)
    hm = (((i0 // Lp) == (i1 // DH)) & ((i0 % Lp) < L_real)).astype(BF16)

    scale = np.float32(1.0 / math.sqrt(DH))
    for li in range(E_LAYERS):
        (wq, bq, wk, bk, wv, bv, wo, bo, w1, b1, w2, b2,
         g1, be1, g2, be2) = (r[...] for r in lrefs[16 * li:16 * li + 16])
        xb = x.astype(BF16)
        qf = jnp.dot(xb, wq.astype(BF16), preferred_element_type=F32) + bq
        kf = jnp.dot(xb, wk.astype(BF16), preferred_element_type=F32) + bk
        vf = jnp.dot(xb, wv.astype(BF16), preferred_element_type=F32) + bv
        rows = []
        for r in range(RB):
            q = (qf[r * Lp:(r + 1) * Lp, :] * scale).astype(BF16)
            k = kf[r * Lp:(r + 1) * Lp, :].astype(BF16)
            v = vf[r * Lp:(r + 1) * Lp, :].astype(BF16)
            km = jnp.concatenate([k] * N_HEADS, axis=0) * hm
            vm = jnp.concatenate([v] * N_HEADS, axis=0) * hm
            s = jax.lax.dot_general(
                q, km, (((1,), (1,)), ((), ())), preferred_element_type=F32)
            p = jnp.exp(s).astype(BF16)
            u = jnp.dot(p, vm, preferred_element_type=F32)
            dexp = jnp.dot(p, hm, preferred_element_type=F32)
            rows.append(u / dexp)
        ao = jnp.concatenate(rows, axis=0).astype(BF16)
        x = x + jnp.dot(ao, wo.astype(BF16), preferred_element_type=F32) + bo
        x = _ln(x, g1, be1)
        y = jnp.dot(x.astype(BF16), w1.astype(BF16),
                    preferred_element_type=F32) + b1
        y = _gelu(y).astype(BF16)
        y = jnp.dot(y, w2.astype(BF16), preferred_element_type=F32) + b2
        x = _ln(x + y, g2, be2)
    return _ln(x, ng_ref[...], nb_ref[...])


def _expert_body(Lp1, In1, Lp2, pn, patch, RB, *refs):
    NW = 16 * E_LAYERS
    x1_ref = refs[0]
    wpe1 = refs[1][...]
    wpe2 = refs[2][...]
    pe1 = refs[3][...]
    pe2 = refs[4][...]
    l1refs = refs[5:5 + NW]
    n1g, n1b = refs[5 + NW], refs[6 + NW]
    l2refs = refs[7 + NW:7 + 2 * NW]
    n2g, n2b = refs[7 + 2 * NW], refs[8 + 2 * NW]
    out_ref = refs[-1]

    x1 = x1_ref[...]
    e1 = jnp.dot(x1, wpe1, preferred_element_type=F32)
    e1 = e1 + jnp.concatenate([pe1] * RB, axis=0)
    e2rows = []
    for r in range(RB):
        x1r = x1[r * Lp1:(r + 1) * Lp1, :]
        e2rows.append(jax.lax.dot_general(
            x1r, wpe2, (((0,), (0,)), ((), ())),
            preferred_element_type=F32) + pe2)
    e2 = jnp.concatenate(e2rows, axis=0)

    o1 = _enc_stack(e1, Lp1, pn, RB, l1refs, n1g, n1b)
    o2 = _enc_stack(e2, Lp2, patch, RB, l2refs, n2g, n2b)

    tt = pn + patch
    for r in range(RB):
        t1 = jnp.transpose(o1[r * Lp1:(r + 1) * Lp1, :])
        t2 = jnp.transpose(o2[r * Lp2:(r + 1) * Lp2, :])
        out_ref[r, :, 0:pn] = t1[:, 0:pn].astype(BF16)
        out_ref[r, :, pn:tt] = t2[:, 0:patch].astype(BF16)


def _pack_layers(layers):
    packed = []
    for p in layers:
        for wname, bname in (("Wq", "bq"), ("Wk", "bk"), ("Wv", "bv"),
                             ("Wo", "bo"), ("W1", "b1"), ("W2", "b2")):
            packed.append(p[wname])
            packed.append(p[bname].reshape(1, -1))
        for g in ("g1", "be1", "g2", "be2"):
            packed.append(p[g].reshape(1, D_MODEL))
    return packed


def _expert_call(x1_2d, wpe1, wpe2, pe1, pe2, l1packed, n1g, n1b,
                 l2packed, n2g, n2b, Lp1, In1, Lp2, pn, patch, RB):
    body = functools.partial(_expert_body, Lp1, In1, Lp2, pn, patch, RB)
    full = lambda a: pl.BlockSpec(a.shape, lambda i: tuple(0 for _ in a.shape))
    args = [x1_2d, wpe1, wpe2, pe1, pe2, *l1packed, n1g, n1b,
            *l2packed, n2g, n2b]
    in_specs = [pl.BlockSpec((RB * Lp1, In1), lambda i: (i, 0))]
    in_specs += [full(a) for a in args[1:]]
    tt = pn + patch
    return pl.pallas_call(
        body,
        grid=(NROWS // RB,),
        in_specs=in_specs,
        out_specs=pl.BlockSpec((RB, D_MODEL, tt), lambda i: (i, 0, 0)),
        out_shape=jax.ShapeDtypeStruct((NROWS, D_MODEL, tt), BF16),
    )(*args)


def _head_body(e0, w0, b0, e1, w1, b1, e2, w2, b2, g_ref, r_ref, out_ref):
    g56 = jnp.dot(r_ref[...], g_ref[...], preferred_element_type=F32)
    acc = (g56[:, 0:1] * (jnp.dot(e0[...], w0[...].astype(BF16),
                                  preferred_element_type=F32) + b0[...])
           + g56[:, 1:2] * (jnp.dot(e1[...], w1[...].astype(BF16),
                                    preferred_element_type=F32) + b1[...])
           + g56[:, 2:3] * (jnp.dot(e2[...], w2[...].astype(BF16),
                                    preferred_element_type=F32) + b2[...]))
    out_ref[...] = acc


def _head_call(ecats, whs, bhs, gates, rmat):
    full0 = lambda a: pl.BlockSpec(a.shape,
                                   lambda: tuple(0 for _ in a.shape))
    in_specs = []
    args = []
    for e in range(NE):
        in_specs += [full0(ecats[e]), full0(whs[e]), full0(bhs[e])]
        args += [ecats[e], whs[e], bhs[e]]
    in_specs += [full0(gates), full0(rmat)]
    args += [gates, rmat]
    return pl.pallas_call(
        _head_body,
        in_specs=in_specs,
        out_specs=pl.BlockSpec((NROWS, PRED_LEN), lambda: (0, 0)),
        out_shape=jax.ShapeDtypeStruct((NROWS, PRED_LEN), F32),
    )(*args)


def _pad_to(a, rows, cols):
    return jnp.pad(a, ((0, rows - a.shape[0]), (0, cols - a.shape[1])))


def kernel(x, params):
    xs = x[..., 0].transpose(0, 2, 1).reshape(NROWS, SEQ_LEN)

    cs = jnp.asarray(_CS_NP)
    wg = _pad_to(params["w_gate"], _KP, 128)
    m = jnp.asarray(_M_NP)
    gates_pad, aux_pad = _gate_call(xs, cs, wg, m)
    aux = aux_pad[0, 0]

    xp = jnp.concatenate(
        [xs, jnp.repeat(xs[:, -1:], PAD, axis=1)], axis=1)
    x8 = xp.reshape(NROWS, (SEQ_LEN + PAD) // STRIDE, STRIDE)

    rb = 8
    ecats = []
    whs = []
    bhs = []
    for e, patch in enumerate(PATCHES):
        p = params["experts"][e]
        pn = _pn_of(patch)
        lp1 = _rup(pn, 8)
        lp2 = _rup(patch, 8)
        inp1 = _rup(patch, 8)
        if patch == 16:
            x1u = jnp.concatenate([x8[:, 0:pn, :], x8[:, 1:pn + 1, :]],
                                  axis=-1)
        else:
            x1u = x8[:, 0:pn, :patch]
        x1 = jnp.pad(x1u, ((0, 0), (0, lp1 - pn), (0, inp1 - patch)))

        wpe1 = _pad_to(p["W_pe1"], inp1, D_MODEL)
        wpe2 = _pad_to(p["W_pe2"], lp1, D_MODEL)
        pe1 = jnp.asarray(np.pad(_PE_NP[:pn], ((0, lp1 - pn), (0, 0))))
        pe2 = jnp.asarray(np.pad(_PE_NP[:patch], ((0, lp2 - patch), (0, 0))))

        oc = _expert_call(x1.reshape(NROWS * lp1, inp1), wpe1, wpe2, pe1, pe2,
                          _pack_layers(p["enc1"]),
                          p["n1"]["g"].reshape(1, D_MODEL),
                          p["n1"]["b"].reshape(1, D_MODEL),
                          _pack_layers(p["enc2"]),
                          p["n2"]["g"].reshape(1, D_MODEL),
                          p["n2"]["b"].reshape(1, D_MODEL),
                          lp1, inp1, lp2, pn, patch, rb)
        ecats.append(oc.reshape(NROWS, D_MODEL * (pn + patch)))
        whs.append(p["Wh"])
        bhs.append(p["bh"].reshape(1, PRED_LEN))

    out56 = _head_call(ecats, whs, bhs, gates_pad, jnp.asarray(_R_NP))
    y = out56.reshape(B, ENC_IN, PRED_LEN).transpose(0, 2, 1)
    return y, aux


# speedup vs baseline: 1.0004x; 1.0004x over previous
import functools
import math

import numpy as np
import jax
import jax.numpy as jnp
from jax.experimental import pallas as pl

SEQ_LEN = 512
PRED_LEN = 96
D_MODEL = 256
N_HEADS = 8
DH = D_MODEL // N_HEADS
E_LAYERS = 2
D_FF = 512
ENC_IN = 7
B = 8
PATCHES = [16, 8, 4]
STRIDE = 8
PAD = 8
NE = 3
NROWS = B * ENC_IN

F32 = jnp.float32
BF16 = jnp.bfloat16


def _pn_of(patch):
    return (SEQ_LEN + PAD - patch) // STRIDE + 1


def _rup(v, m):
    return ((v + m - 1) // m) * m


def _sinusoid_pe(max_len, d):
    pe = np.zeros((max_len, d), dtype=np.float32)
    pos = np.arange(max_len, dtype=np.float32)[:, None]
    div = np.exp(np.arange(0, d, 2, dtype=np.float32) * (-np.log(10000.0) / d))
    pe[:, 0::2] = np.sin(pos * div)
    pe[:, 1::2] = np.cos(pos * div)
    return pe


_PE_NP = _sinusoid_pe(128, D_MODEL)

_KF = SEQ_LEN // 2 + 1
_KP = 384
_n = np.arange(SEQ_LEN, dtype=np.float64)[:, None]
_k = np.arange(_KF, dtype=np.float64)[None, :]
_ang = 2.0 * np.pi * _n * _k / SEQ_LEN
_COS = np.zeros((SEQ_LEN, _KP), dtype=np.float32)
_SIN = np.zeros((SEQ_LEN, _KP), dtype=np.float32)
_COS[:, :_KF] = np.cos(_ang).astype(np.float32)
_SIN[:, :_KF] = np.sin(_ang).astype(np.float32)
_CS_NP = np.concatenate([_COS, _SIN], axis=1)

_M_NP = np.zeros((B, NROWS), dtype=np.float32)
for _b in range(B):
    _M_NP[_b, _b * ENC_IN:(_b + 1) * ENC_IN] = 1.0 / ENC_IN

_R_NP = np.zeros((NROWS, B), dtype=np.float32)
for _i in range(NROWS):
    _R_NP[_i, _i // ENC_IN] = 1.0


def _ln(x, g, b):
    m = jnp.mean(x, axis=-1, keepdims=True)
    v = jnp.mean((x - m) ** 2, axis=-1, keepdims=True)
    return (x - m) / jnp.sqrt(v + 1e-5) * g + b


def _gelu(x):
    return 0.5 * x * (1.0 + jax.lax.erf(x / np.float32(math.sqrt(2.0))))


def _gate_body(x_ref, cs_ref, wg_ref, m_ref, gates_ref, aux_ref):
    ri = jnp.dot(x_ref[...], cs_ref[...], preferred_element_type=F32)
    re = ri[:, :_KP]
    im = ri[:, _KP:]
    amp = jnp.sqrt(re * re + im * im)
    l56 = jnp.dot(amp, wg_ref[...], preferred_element_type=F32)
    lg = jnp.dot(m_ref[...], l56, preferred_element_type=F32)
    l0 = lg[:, 0:1]
    l1 = lg[:, 1:2]
    l2 = lg[:, 2:3]
    c01 = l0 >= l1
    c02 = l0 >= l2
    c12 = l1 >= l2
    i0_is0 = c01 & c02
    i0_is1 = (~i0_is0) & c12
    i0_is2 = (~i0_is0) & (~c12)
    v0 = jnp.where(i0_is0, l0, jnp.where(i0_is1, l1, l2))
    i1_is0 = (i0_is1 & c02) | (i0_is2 & c01)
    i1_is1 = (i0_is0 & c12) | (i0_is2 & (~c01))
    i1_is2 = (i0_is0 & (~c12)) | (i0_is1 & (~c02))
    v1 = jnp.where(i1_is0, l0, jnp.where(i1_is1, l1, l2))
    e1w = jnp.exp(v1 - v0)
    denom = 1.0 + e1w
    gf = 1.0 / denom
    gs = e1w / denom
    zero = jnp.zeros_like(gf)
    g_cols = []
    for e, (a, bb) in enumerate(((i0_is0, i1_is0), (i0_is1, i1_is1),
                                 (i0_is2, i1_is2))):
        g_cols.append(jnp.where(a, gf, zero) + jnp.where(bb, gs, zero))
    gates_ref[...] = jnp.zeros((B, 128), F32)
    imp = []
    load = []
    for e in range(NE):
        gates_ref[:, e:e + 1] = g_cols[e]
        imp.append(jnp.sum(g_cols[e]))
        load.append(jnp.sum((g_cols[e] > 0).astype(F32)))

    def cv3(a, bb, c):
        m = (a + bb + c) / 3.0
        var = ((a - m) ** 2 + (bb - m) ** 2 + (c - m) ** 2) / 2.0
        return var / (m * m + 1e-10)

    aux = (cv3(*imp) + cv3(*load)) * 0.01
    aux_ref[...] = jnp.broadcast_to(aux, (1, 128)).astype(F32)


def _gate_call(xs, cs, wg, m):
    return pl.pallas_call(
        _gate_body,
        grid=(1,),
        in_specs=[
            pl.BlockSpec(xs.shape, lambda i: (0, 0)),
            pl.BlockSpec(cs.shape, lambda i: (0, 0)),
            pl.BlockSpec(wg.shape, lambda i: (0, 0)),
            pl.BlockSpec(m.shape, lambda i: (0, 0)),
        ],
        out_specs=[
            pl.BlockSpec((B, 128), lambda i: (0, 0)),
            pl.BlockSpec((1, 128), lambda i: (0, 0)),
        ],
        out_shape=[
            jax.ShapeDtypeStruct((B, 128), F32),
            jax.ShapeDtypeStruct((1, 128), F32),
        ],
    )(xs, cs, wg, m)


def _enc_stack(x, Lp, L_real, RB, lrefs, ng_ref, nb_ref):
    i0 = jax.lax.broadcasted_iota(jnp.int32, (N_HEADS * Lp, D_MODEL), 0)
    i1 = jax.lax.broadcasted_iota(jnp.int32, (N_HEADS * Lp, D_MODEL), 1)
    hm = (((i0 // Lp) == (i1 // DH)) & ((i0 % Lp) < L_real)).astype(BF16)

    scale = np.float32(1.0 / math.sqrt(DH))
    for li in range(E_LAYERS):
        (wq, bq, wk, bk, wv, bv, wo, bo, w1, b1, w2, b2,
         g1, be1, g2, be2) = (r[...] for r in lrefs[16 * li:16 * li + 16])
        xb = x.astype(BF16)
        qf = jnp.dot(xb, wq.astype(BF16), preferred_element_type=F32) + bq
        kf = jnp.dot(xb, wk.astype(BF16), preferred_element_type=F32) + bk
        vf = jnp.dot(xb, wv.astype(BF16), preferred_element_type=F32) + bv
        rows = []
        for r in range(RB):
            q = (qf[r * Lp:(r + 1) * Lp, :] * scale).astype(BF16)
            k = kf[r * Lp:(r + 1) * Lp, :].astype(BF16)
            v = vf[r * Lp:(r + 1) * Lp, :].astype(BF16)
            km = jnp.concatenate([k] * N_HEADS, axis=0) * hm
            vm = jnp.concatenate([v] * N_HEADS, axis=0) * hm
            s = jax.lax.dot_general(
                q, km, (((1,), (1,)), ((), ())), preferred_element_type=F32)
            p = jnp.exp(s).astype(BF16)
            u = jnp.dot(p, vm, preferred_element_type=F32)
            dexp = jnp.dot(p, hm, preferred_element_type=F32)
            rows.append(u / dexp)
        ao = jnp.concatenate(rows, axis=0).astype(BF16)
        x = x + jnp.dot(ao, wo.astype(BF16), preferred_element_type=F32) + bo
        x = _ln(x, g1, be1)
        y = jnp.dot(x.astype(BF16), w1.astype(BF16),
                    preferred_element_type=F32) + b1
        y = _gelu(y).astype(BF16)
        y = jnp.dot(y, w2.astype(BF16), preferred_element_type=F32) + b2
        x = _ln(x + y, g2, be2)
    return _ln(x, ng_ref[...], nb_ref[...])


def _expert_body(Lp1, In1, Lp2, pn, patch, RB, *refs):
    NW = 16 * E_LAYERS
    x1_ref = refs[0]
    wpe1 = refs[1][...]
    wpe2 = refs[2][...]
    pe1 = refs[3][...]
    pe2 = refs[4][...]
    l1refs = refs[5:5 + NW]
    n1g, n1b = refs[5 + NW], refs[6 + NW]
    l2refs = refs[7 + NW:7 + 2 * NW]
    n2g, n2b = refs[7 + 2 * NW], refs[8 + 2 * NW]
    out_ref = refs[-1]

    x1 = x1_ref[...]
    e1 = jnp.dot(x1, wpe1, preferred_element_type=F32)
    e1 = e1 + jnp.concatenate([pe1] * RB, axis=0)
    e2rows = []
    for r in range(RB):
        x1r = x1[r * Lp1:(r + 1) * Lp1, :]
        e2rows.append(jax.lax.dot_general(
            x1r, wpe2, (((0,), (0,)), ((), ())),
            preferred_element_type=F32) + pe2)
    e2 = jnp.concatenate(e2rows, axis=0)

    o1 = _enc_stack(e1, Lp1, pn, RB, l1refs, n1g, n1b)
    o2 = _enc_stack(e2, Lp2, patch, RB, l2refs, n2g, n2b)

    tt = pn + patch
    for r in range(RB):
        t1 = jnp.transpose(o1[r * Lp1:(r + 1) * Lp1, :])
        t2 = jnp.transpose(o2[r * Lp2:(r + 1) * Lp2, :])
        out_ref[r, :, 0:pn] = t1[:, 0:pn]
        out_ref[r, :, pn:tt] = t2[:, 0:patch]


def _pack_layers(layers):
    packed = []
    for p in layers:
        for wname, bname in (("Wq", "bq"), ("Wk", "bk"), ("Wv", "bv"),
                             ("Wo", "bo"), ("W1", "b1"), ("W2", "b2")):
            packed.append(p[wname])
            packed.append(p[bname].reshape(1, -1))
        for g in ("g1", "be1", "g2", "be2"):
            packed.append(p[g].reshape(1, D_MODEL))
    return packed


def _expert_call(x1_2d, wpe1, wpe2, pe1, pe2, l1packed, n1g, n1b,
                 l2packed, n2g, n2b, Lp1, In1, Lp2, pn, patch, RB):
    body = functools.partial(_expert_body, Lp1, In1, Lp2, pn, patch, RB)
    full = lambda a: pl.BlockSpec(a.shape, lambda i: tuple(0 for _ in a.shape))
    args = [x1_2d, wpe1, wpe2, pe1, pe2, *l1packed, n1g, n1b,
            *l2packed, n2g, n2b]
    in_specs = [pl.BlockSpec((RB * Lp1, In1), lambda i: (i, 0))]
    in_specs += [full(a) for a in args[1:]]
    tt = pn + patch
    return pl.pallas_call(
        body,
        grid=(NROWS // RB,),
        in_specs=in_specs,
        out_specs=pl.BlockSpec((RB, D_MODEL, tt), lambda i: (i, 0, 0)),
        out_shape=jax.ShapeDtypeStruct((NROWS, D_MODEL, tt), F32),
    )(*args)


def _head_body(e0, w0, b0, e1, w1, b1, e2, w2, b2, g_ref, r_ref, out_ref):
    g56 = jnp.dot(r_ref[...], g_ref[...], preferred_element_type=F32)
    acc = (g56[:, 0:1] * (jnp.dot(e0[...], w0[...],
                                  preferred_element_type=F32) + b0[...])
           + g56[:, 1:2] * (jnp.dot(e1[...], w1[...],
                                    preferred_element_type=F32) + b1[...])
           + g56[:, 2:3] * (jnp.dot(e2[...], w2[...],
                                    preferred_element_type=F32) + b2[...]))
    out_ref[...] = acc


def _head_call(ecats, whs, bhs, gates, rmat):
    full0 = lambda a: pl.BlockSpec(a.shape,
                                   lambda: tuple(0 for _ in a.shape))
    in_specs = []
    args = []
    for e in range(NE):
        in_specs += [full0(ecats[e]), full0(whs[e]), full0(bhs[e])]
        args += [ecats[e], whs[e], bhs[e]]
    in_specs += [full0(gates), full0(rmat)]
    args += [gates, rmat]
    return pl.pallas_call(
        _head_body,
        in_specs=in_specs,
        out_specs=pl.BlockSpec((NROWS, PRED_LEN), lambda: (0, 0)),
        out_shape=jax.ShapeDtypeStruct((NROWS, PRED_LEN), F32),
    )(*args)


def _pad_to(a, rows, cols):
    return jnp.pad(a, ((0, rows - a.shape[0]), (0, cols - a.shape[1])))


def kernel(x, params):
    xs = x[..., 0].transpose(0, 2, 1).reshape(NROWS, SEQ_LEN)

    cs = jnp.asarray(_CS_NP)
    wg = _pad_to(params["w_gate"], _KP, 128)
    m = jnp.asarray(_M_NP)
    gates_pad, aux_pad = _gate_call(xs, cs, wg, m)
    aux = aux_pad[0, 0]

    xp = jnp.concatenate(
        [xs, jnp.repeat(xs[:, -1:], PAD, axis=1)], axis=1)
    x8 = xp.reshape(NROWS, (SEQ_LEN + PAD) // STRIDE, STRIDE)

    rb = 8
    ecats = []
    whs = []
    bhs = []
    for e, patch in enumerate(PATCHES):
        p = params["experts"][e]
        pn = _pn_of(patch)
        lp1 = _rup(pn, 8)
        lp2 = _rup(patch, 8)
        inp1 = _rup(patch, 8)
        if patch == 16:
            x1u = jnp.concatenate([x8[:, 0:pn, :], x8[:, 1:pn + 1, :]],
                                  axis=-1)
        else:
            x1u = x8[:, 0:pn, :patch]
        x1 = jnp.pad(x1u, ((0, 0), (0, lp1 - pn), (0, inp1 - patch)))

        wpe1 = _pad_to(p["W_pe1"], inp1, D_MODEL)
        wpe2 = _pad_to(p["W_pe2"], lp1, D_MODEL)
        pe1 = jnp.asarray(np.pad(_PE_NP[:pn], ((0, lp1 - pn), (0, 0))))
        pe2 = jnp.asarray(np.pad(_PE_NP[:patch], ((0, lp2 - patch), (0, 0))))

        oc = _expert_call(x1.reshape(NROWS * lp1, inp1), wpe1, wpe2, pe1, pe2,
                          _pack_layers(p["enc1"]),
                          p["n1"]["g"].reshape(1, D_MODEL),
                          p["n1"]["b"].reshape(1, D_MODEL),
                          _pack_layers(p["enc2"]),
                          p["n2"]["g"].reshape(1, D_MODEL),
                          p["n2"]["b"].reshape(1, D_MODEL),
                          lp1, inp1, lp2, pn, patch, rb)
        ecats.append(oc.reshape(NROWS, D_MODEL * (pn + patch)))
        whs.append(p["Wh"])
        bhs.append(p["bh"].reshape(1, PRED_LEN))

    out56 = _head_call(ecats, whs, bhs, gates_pad, jnp.asarray(_R_NP))
    y = out56.reshape(B, ENC_IN, PRED_LEN).transpose(0, 2, 1)
    return y, aux


# speedup vs baseline: 1.0397x; 1.0393x over previous
import functools
import math

import numpy as np
import jax
import jax.numpy as jnp
from jax.experimental import pallas as pl

SEQ_LEN = 512
PRED_LEN = 96
D_MODEL = 256
N_HEADS = 8
DH = D_MODEL // N_HEADS
E_LAYERS = 2
D_FF = 512
ENC_IN = 7
B = 8
PATCHES = [16, 8, 4]
STRIDE = 8
PAD = 8
NE = 3
NROWS = B * ENC_IN

F32 = jnp.float32
BF16 = jnp.bfloat16


def _pn_of(patch):
    return (SEQ_LEN + PAD - patch) // STRIDE + 1


def _rup(v, m):
    return ((v + m - 1) // m) * m


def _sinusoid_pe(max_len, d):
    pe = np.zeros((max_len, d), dtype=np.float32)
    pos = np.arange(max_len, dtype=np.float32)[:, None]
    div = np.exp(np.arange(0, d, 2, dtype=np.float32) * (-np.log(10000.0) / d))
    pe[:, 0::2] = np.sin(pos * div)
    pe[:, 1::2] = np.cos(pos * div)
    return pe


_PE_NP = _sinusoid_pe(128, D_MODEL)

_KF = SEQ_LEN // 2 + 1
_KP = 384
_n = np.arange(SEQ_LEN, dtype=np.float64)[:, None]
_k = np.arange(_KF, dtype=np.float64)[None, :]
_ang = 2.0 * np.pi * _n * _k / SEQ_LEN
_COS = np.zeros((SEQ_LEN, _KP), dtype=np.float32)
_SIN = np.zeros((SEQ_LEN, _KP), dtype=np.float32)
_COS[:, :_KF] = np.cos(_ang).astype(np.float32)
_SIN[:, :_KF] = np.sin(_ang).astype(np.float32)
_CS_NP = np.concatenate([_COS, _SIN], axis=1)

_M_NP = np.zeros((B, NROWS), dtype=np.float32)
for _b in range(B):
    _M_NP[_b, _b * ENC_IN:(_b + 1) * ENC_IN] = 1.0 / ENC_IN

_R_NP = np.zeros((NROWS, B), dtype=np.float32)
for _i in range(NROWS):
    _R_NP[_i, _i // ENC_IN] = 1.0


def _ln(x, g, b):
    m = jnp.mean(x, axis=-1, keepdims=True)
    v = jnp.mean((x - m) ** 2, axis=-1, keepdims=True)
    return (x - m) / jnp.sqrt(v + 1e-5) * g + b


def _gelu(x):
    return 0.5 * x * (1.0 + jax.lax.erf(x / np.float32(math.sqrt(2.0))))


def _gate_body(x_ref, cs_ref, wg_ref, m_ref, gates_ref, aux_ref):
    ri = jnp.dot(x_ref[...], cs_ref[...], preferred_element_type=F32)
    re = ri[:, :_KP]
    im = ri[:, _KP:]
    amp = jnp.sqrt(re * re + im * im)
    l56 = jnp.dot(amp, wg_ref[...], preferred_element_type=F32)
    lg = jnp.dot(m_ref[...], l56, preferred_element_type=F32)
    l0 = lg[:, 0:1]
    l1 = lg[:, 1:2]
    l2 = lg[:, 2:3]
    c01 = l0 >= l1
    c02 = l0 >= l2
    c12 = l1 >= l2
    i0_is0 = c01 & c02
    i0_is1 = (~i0_is0) & c12
    i0_is2 = (~i0_is0) & (~c12)
    v0 = jnp.where(i0_is0, l0, jnp.where(i0_is1, l1, l2))
    i1_is0 = (i0_is1 & c02) | (i0_is2 & c01)
    i1_is1 = (i0_is0 & c12) | (i0_is2 & (~c01))
    i1_is2 = (i0_is0 & (~c12)) | (i0_is1 & (~c02))
    v1 = jnp.where(i1_is0, l0, jnp.where(i1_is1, l1, l2))
    e1w = jnp.exp(v1 - v0)
    denom = 1.0 + e1w
    gf = 1.0 / denom
    gs = e1w / denom
    zero = jnp.zeros_like(gf)
    g_cols = []
    for e, (a, bb) in enumerate(((i0_is0, i1_is0), (i0_is1, i1_is1),
                                 (i0_is2, i1_is2))):
        g_cols.append(jnp.where(a, gf, zero) + jnp.where(bb, gs, zero))
    gates_ref[...] = jnp.zeros((B, 128), F32)
    imp = []
    load = []
    for e in range(NE):
        gates_ref[:, e:e + 1] = g_cols[e]
        imp.append(jnp.sum(g_cols[e]))
        load.append(jnp.sum((g_cols[e] > 0).astype(F32)))

    def cv3(a, bb, c):
        m = (a + bb + c) / 3.0
        var = ((a - m) ** 2 + (bb - m) ** 2 + (c - m) ** 2) / 2.0
        return var / (m * m + 1e-10)

    aux = (cv3(*imp) + cv3(*load)) * 0.01
    aux_ref[...] = jnp.broadcast_to(aux, (1, 128)).astype(F32)


def _gate_call(xs, cs, wg, m):
    return pl.pallas_call(
        _gate_body,
        grid=(1,),
        in_specs=[
            pl.BlockSpec(xs.shape, lambda i: (0, 0)),
            pl.BlockSpec(cs.shape, lambda i: (0, 0)),
            pl.BlockSpec(wg.shape, lambda i: (0, 0)),
            pl.BlockSpec(m.shape, lambda i: (0, 0)),
        ],
        out_specs=[
            pl.BlockSpec((B, 128), lambda i: (0, 0)),
            pl.BlockSpec((1, 128), lambda i: (0, 0)),
        ],
        out_shape=[
            jax.ShapeDtypeStruct((B, 128), F32),
            jax.ShapeDtypeStruct((1, 128), F32),
        ],
    )(xs, cs, wg, m)


def _enc_stack(x, Lp, L_real, RB, lrefs, ng_ref, nb_ref):
    i0 = jax.lax.broadcasted_iota(jnp.int32, (N_HEADS * Lp, D_MODEL), 0)
    i1 = jax.lax.broadcasted_iota(jnp.int32, (N_HEADS * Lp, D_MODEL), 1)
    hm = (((i0 // Lp) == (i1 // DH)) & ((i0 % Lp) < L_real)).astype(BF16)

    scale = np.float32(1.0 / math.sqrt(DH))
    for li in range(E_LAYERS):
        (wq, bq, wk, bk, wv, bv, wo, bo, w1, b1, w2, b2,
         g1, be1, g2, be2) = (r[...] for r in lrefs[16 * li:16 * li + 16])
        qf = jnp.dot(x, wq, preferred_element_type=F32) + bq
        kf = jnp.dot(x, wk, preferred_element_type=F32) + bk
        vf = jnp.dot(x, wv, preferred_element_type=F32) + bv
        rows = []
        for r in range(RB):
            q = (qf[r * Lp:(r + 1) * Lp, :] * scale).astype(BF16)
            k = kf[r * Lp:(r + 1) * Lp, :].astype(BF16)
            v = vf[r * Lp:(r + 1) * Lp, :].astype(BF16)
            km = jnp.concatenate([k] * N_HEADS, axis=0) * hm
            vm = jnp.concatenate([v] * N_HEADS, axis=0) * hm
            s = jax.lax.dot_general(
                q, km, (((1,), (1,)), ((), ())), preferred_element_type=F32)
            p = jnp.exp(s).astype(BF16)
            u = jnp.dot(p, vm, preferred_element_type=F32)
            dexp = jnp.dot(p, hm, preferred_element_type=F32)
            rows.append(u / dexp)
        ao = jnp.concatenate(rows, axis=0)
        x = x + jnp.dot(ao, wo, preferred_element_type=F32) + bo
        x = _ln(x, g1, be1)
        y = jnp.dot(x, w1, preferred_element_type=F32) + b1
        y = _gelu(y)
        y = jnp.dot(y, w2, preferred_element_type=F32) + b2
        x = _ln(x + y, g2, be2)
    return _ln(x, ng_ref[...], nb_ref[...])


def _expert_body(Lp1, In1, Lp2, pn, patch, RB, *refs):
    NW = 16 * E_LAYERS
    x1_ref = refs[0]
    wpe1 = refs[1][...]
    wpe2 = refs[2][...]
    pe1 = refs[3][...]
    pe2 = refs[4][...]
    l1refs = refs[5:5 + NW]
    n1g, n1b = refs[5 + NW], refs[6 + NW]
    l2refs = refs[7 + NW:7 + 2 * NW]
    n2g, n2b = refs[7 + 2 * NW], refs[8 + 2 * NW]
    out_ref = refs[-1]

    x1 = x1_ref[...]
    e1 = jnp.dot(x1, wpe1, preferred_element_type=F32)
    e1 = e1 + jnp.concatenate([pe1] * RB, axis=0)
    e2rows = []
    for r in range(RB):
        x1r = x1[r * Lp1:(r + 1) * Lp1, :]
        e2rows.append(jax.lax.dot_general(
            x1r, wpe2, (((0,), (0,)), ((), ())),
            preferred_element_type=F32) + pe2)
    e2 = jnp.concatenate(e2rows, axis=0)

    o1 = _enc_stack(e1, Lp1, pn, RB, l1refs, n1g, n1b)
    o2 = _enc_stack(e2, Lp2, patch, RB, l2refs, n2g, n2b)

    tt = pn + patch
    for r in range(RB):
        t1 = jnp.transpose(o1[r * Lp1:(r + 1) * Lp1, :])
        t2 = jnp.transpose(o2[r * Lp2:(r + 1) * Lp2, :])
        out_ref[r, :, 0:pn] = t1[:, 0:pn]
        out_ref[r, :, pn:tt] = t2[:, 0:patch]


def _pack_layers(layers):
    packed = []
    for p in layers:
        for wname, bname in (("Wq", "bq"), ("Wk", "bk"), ("Wv", "bv"),
                             ("Wo", "bo"), ("W1", "b1"), ("W2", "b2")):
            packed.append(p[wname])
            packed.append(p[bname].reshape(1, -1))
        for g in ("g1", "be1", "g2", "be2"):
            packed.append(p[g].reshape(1, D_MODEL))
    return packed


def _expert_call(x1_2d, wpe1, wpe2, pe1, pe2, l1packed, n1g, n1b,
                 l2packed, n2g, n2b, Lp1, In1, Lp2, pn, patch, RB):
    body = functools.partial(_expert_body, Lp1, In1, Lp2, pn, patch, RB)
    full = lambda a: pl.BlockSpec(a.shape, lambda i: tuple(0 for _ in a.shape))
    args = [x1_2d, wpe1, wpe2, pe1, pe2, *l1packed, n1g, n1b,
            *l2packed, n2g, n2b]
    in_specs = [pl.BlockSpec((RB * Lp1, In1), lambda i: (i, 0))]
    in_specs += [full(a) for a in args[1:]]
    tt = pn + patch
    return pl.pallas_call(
        body,
        grid=(NROWS // RB,),
        in_specs=in_specs,
        out_specs=pl.BlockSpec((RB, D_MODEL, tt), lambda i: (i, 0, 0)),
        out_shape=jax.ShapeDtypeStruct((NROWS, D_MODEL, tt), F32),
    )(*args)


def _head_body(e0, w0, b0, e1, w1, b1, e2, w2, b2, g_ref, r_ref, out_ref):
    kb = pl.program_id(0)
    g56 = jnp.dot(r_ref[...], g_ref[...], preferred_element_type=F32)

    @pl.when(kb == 0)
    def _init():
        out_ref[...] = (g56[:, 0:1] * b0[...] + g56[:, 1:2] * b1[...]
                        + g56[:, 2:3] * b2[...])

    acc = (g56[:, 0:1] * jnp.dot(e0[...], w0[...],
                                 preferred_element_type=F32)
           + g56[:, 1:2] * jnp.dot(e1[...], w1[...],
                                   preferred_element_type=F32)
           + g56[:, 2:3] * jnp.dot(e2[...], w2[...],
                                   preferred_element_type=F32))
    out_ref[...] = out_ref[...] + acc


_HEAD_NB = 2


def _head_call(ecats, whs, bhs, gates, rmat):
    full0 = lambda a: pl.BlockSpec(a.shape,
                                   lambda kb: tuple(0 for _ in a.shape))
    in_specs = []
    args = []
    for e in range(NE):
        kb_e = ecats[e].shape[1] // _HEAD_NB
        in_specs.append(pl.BlockSpec((NROWS, kb_e), lambda kb: (0, kb)))
        in_specs.append(pl.BlockSpec((kb_e, PRED_LEN), lambda kb: (kb, 0)))
        in_specs.append(full0(bhs[e]))
        args += [ecats[e], whs[e], bhs[e]]
    in_specs += [full0(gates), full0(rmat)]
    args += [gates, rmat]
    return pl.pallas_call(
        _head_body,
        grid=(_HEAD_NB,),
        in_specs=in_specs,
        out_specs=pl.BlockSpec((NROWS, PRED_LEN), lambda kb: (0, 0)),
        out_shape=jax.ShapeDtypeStruct((NROWS, PRED_LEN), F32),
    )(*args)


def _pad_to(a, rows, cols):
    return jnp.pad(a, ((0, rows - a.shape[0]), (0, cols - a.shape[1])))


def kernel(x, params):
    xs = x[..., 0].transpose(0, 2, 1).reshape(NROWS, SEQ_LEN)

    cs = jnp.asarray(_CS_NP)
    wg = _pad_to(params["w_gate"], _KP, 128)
    m = jnp.asarray(_M_NP)
    gates_pad, aux_pad = _gate_call(xs, cs, wg, m)
    aux = aux_pad[0, 0]

    xp = jnp.concatenate(
        [xs, jnp.repeat(xs[:, -1:], PAD, axis=1)], axis=1)
    x8 = xp.reshape(NROWS, (SEQ_LEN + PAD) // STRIDE, STRIDE)

    rb = 8
    ecats = []
    whs = []
    bhs = []
    for e, patch in enumerate(PATCHES):
        p = params["experts"][e]
        pn = _pn_of(patch)
        lp1 = _rup(pn, 8)
        lp2 = _rup(patch, 8)
        inp1 = _rup(patch, 8)
        if patch == 16:
            x1u = jnp.concatenate([x8[:, 0:pn, :], x8[:, 1:pn + 1, :]],
                                  axis=-1)
        else:
            x1u = x8[:, 0:pn, :patch]
        x1 = jnp.pad(x1u, ((0, 0), (0, lp1 - pn), (0, inp1 - patch)))

        wpe1 = _pad_to(p["W_pe1"], inp1, D_MODEL)
        wpe2 = _pad_to(p["W_pe2"], lp1, D_MODEL)
        pe1 = jnp.asarray(np.pad(_PE_NP[:pn], ((0, lp1 - pn), (0, 0))))
        pe2 = jnp.asarray(np.pad(_PE_NP[:patch], ((0, lp2 - patch), (0, 0))))

        oc = _expert_call(x1.reshape(NROWS * lp1, inp1), wpe1, wpe2, pe1, pe2,
                          _pack_layers(p["enc1"]),
                          p["n1"]["g"].reshape(1, D_MODEL),
                          p["n1"]["b"].reshape(1, D_MODEL),
                          _pack_layers(p["enc2"]),
                          p["n2"]["g"].reshape(1, D_MODEL),
                          p["n2"]["b"].reshape(1, D_MODEL),
                          lp1, inp1, lp2, pn, patch, rb)
        ecats.append(oc.reshape(NROWS, D_MODEL * (pn + patch)))
        whs.append(p["Wh"])
        bhs.append(p["bh"].reshape(1, PRED_LEN))

    out56 = _head_call(ecats, whs, bhs, gates_pad, jnp.asarray(_R_NP))
    y = out56.reshape(B, ENC_IN, PRED_LEN).transpose(0, 2, 1)
    return y, aux


# speedup vs baseline: 1.1480x; 1.1041x over previous
import functools
import math

import numpy as np
import jax
import jax.numpy as jnp
from jax.experimental import pallas as pl

SEQ_LEN = 512
PRED_LEN = 96
D_MODEL = 256
N_HEADS = 8
DH = D_MODEL // N_HEADS
E_LAYERS = 2
D_FF = 512
ENC_IN = 7
B = 8
PATCHES = [16, 8, 4]
STRIDE = 8
PAD = 8
NE = 3
NROWS = B * ENC_IN

F32 = jnp.float32
BF16 = jnp.bfloat16


def _pn_of(patch):
    return (SEQ_LEN + PAD - patch) // STRIDE + 1


def _rup(v, m):
    return ((v + m - 1) // m) * m


def _sinusoid_pe(max_len, d):
    pe = np.zeros((max_len, d), dtype=np.float32)
    pos = np.arange(max_len, dtype=np.float32)[:, None]
    div = np.exp(np.arange(0, d, 2, dtype=np.float32) * (-np.log(10000.0) / d))
    pe[:, 0::2] = np.sin(pos * div)
    pe[:, 1::2] = np.cos(pos * div)
    return pe


_PE_NP = _sinusoid_pe(128, D_MODEL)

_KF = SEQ_LEN // 2 + 1
_KP = 384
_n = np.arange(SEQ_LEN, dtype=np.float64)[:, None]
_k = np.arange(_KF, dtype=np.float64)[None, :]
_ang = 2.0 * np.pi * _n * _k / SEQ_LEN
_COS = np.zeros((SEQ_LEN, _KP), dtype=np.float32)
_SIN = np.zeros((SEQ_LEN, _KP), dtype=np.float32)
_COS[:, :_KF] = np.cos(_ang).astype(np.float32)
_SIN[:, :_KF] = np.sin(_ang).astype(np.float32)
_CS_NP = np.concatenate([_COS, _SIN], axis=1)

_M_NP = np.zeros((B, NROWS), dtype=np.float32)
for _b in range(B):
    _M_NP[_b, _b * ENC_IN:(_b + 1) * ENC_IN] = 1.0 / ENC_IN

_R_NP = np.zeros((NROWS, B), dtype=np.float32)
for _i in range(NROWS):
    _R_NP[_i, _i // ENC_IN] = 1.0


def _ln(x, g, b):
    m = jnp.mean(x, axis=-1, keepdims=True)
    v = jnp.mean((x - m) ** 2, axis=-1, keepdims=True)
    return (x - m) / jnp.sqrt(v + 1e-5) * g + b


def _gelu(x):
    return 0.5 * x * (1.0 + jax.lax.erf(x / np.float32(math.sqrt(2.0))))


def _gate_body(x_ref, cs_ref, wg_ref, m_ref, gates_ref, aux_ref):
    ri = jnp.dot(x_ref[...], cs_ref[...], preferred_element_type=F32)
    re = ri[:, :_KP]
    im = ri[:, _KP:]
    amp = jnp.sqrt(re * re + im * im)
    l56 = jnp.dot(amp, wg_ref[...], preferred_element_type=F32)
    lg = jnp.dot(m_ref[...], l56, preferred_element_type=F32)
    l0 = lg[:, 0:1]
    l1 = lg[:, 1:2]
    l2 = lg[:, 2:3]
    c01 = l0 >= l1
    c02 = l0 >= l2
    c12 = l1 >= l2
    i0_is0 = c01 & c02
    i0_is1 = (~i0_is0) & c12
    i0_is2 = (~i0_is0) & (~c12)
    v0 = jnp.where(i0_is0, l0, jnp.where(i0_is1, l1, l2))
    i1_is0 = (i0_is1 & c02) | (i0_is2 & c01)
    i1_is1 = (i0_is0 & c12) | (i0_is2 & (~c01))
    i1_is2 = (i0_is0 & (~c12)) | (i0_is1 & (~c02))
    v1 = jnp.where(i1_is0, l0, jnp.where(i1_is1, l1, l2))
    e1w = jnp.exp(v1 - v0)
    denom = 1.0 + e1w
    gf = 1.0 / denom
    gs = e1w / denom
    zero = jnp.zeros_like(gf)
    g_cols = []
    for e, (a, bb) in enumerate(((i0_is0, i1_is0), (i0_is1, i1_is1),
                                 (i0_is2, i1_is2))):
        g_cols.append(jnp.where(a, gf, zero) + jnp.where(bb, gs, zero))
    gates_ref[...] = jnp.zeros((B, 128), F32)
    imp = []
    load = []
    for e in range(NE):
        gates_ref[:, e:e + 1] = g_cols[e]
        imp.append(jnp.sum(g_cols[e]))
        load.append(jnp.sum((g_cols[e] > 0).astype(F32)))

    def cv3(a, bb, c):
        m = (a + bb + c) / 3.0
        var = ((a - m) ** 2 + (bb - m) ** 2 + (c - m) ** 2) / 2.0
        return var / (m * m + 1e-10)

    aux = (cv3(*imp) + cv3(*load)) * 0.01
    aux_ref[...] = jnp.broadcast_to(aux, (1, 128)).astype(F32)


def _gate_call(xs, cs, wg, m):
    return pl.pallas_call(
        _gate_body,
        grid=(1,),
        in_specs=[
            pl.BlockSpec(xs.shape, lambda i: (0, 0)),
            pl.BlockSpec(cs.shape, lambda i: (0, 0)),
            pl.BlockSpec(wg.shape, lambda i: (0, 0)),
            pl.BlockSpec(m.shape, lambda i: (0, 0)),
        ],
        out_specs=[
            pl.BlockSpec((B, 128), lambda i: (0, 0)),
            pl.BlockSpec((1, 128), lambda i: (0, 0)),
        ],
        out_shape=[
            jax.ShapeDtypeStruct((B, 128), F32),
            jax.ShapeDtypeStruct((1, 128), F32),
        ],
    )(xs, cs, wg, m)


def _enc_stack(x, Lp, L_real, RB, lrefs, ng_ref, nb_ref):
    i0 = jax.lax.broadcasted_iota(jnp.int32, (N_HEADS * Lp, D_MODEL), 0)
    i1 = jax.lax.broadcasted_iota(jnp.int32, (N_HEADS * Lp, D_MODEL), 1)
    hm = (((i0 // Lp) == (i1 // DH)) & ((i0 % Lp) < L_real)).astype(BF16)

    scale = np.float32(1.0 / math.sqrt(DH))
    for li in range(E_LAYERS):
        (wq, bq, wk, bk, wv, bv, wo, bo, w1, b1, w2, b2,
         g1, be1, g2, be2) = (r[...] for r in lrefs[16 * li:16 * li + 16])
        qf = jnp.dot(x, wq, preferred_element_type=F32) + bq
        kf = jnp.dot(x, wk, preferred_element_type=F32) + bk
        vf = jnp.dot(x, wv, preferred_element_type=F32) + bv
        rows = []
        for r in range(RB):
            q = (qf[r * Lp:(r + 1) * Lp, :] * scale).astype(BF16)
            k = kf[r * Lp:(r + 1) * Lp, :].astype(BF16)
            v = vf[r * Lp:(r + 1) * Lp, :].astype(BF16)
            km = jnp.concatenate([k] * N_HEADS, axis=0) * hm
            vm = jnp.concatenate([v] * N_HEADS, axis=0) * hm
            s = jax.lax.dot_general(
                q, km, (((1,), (1,)), ((), ())), preferred_element_type=F32)
            p = jnp.exp(s).astype(BF16)
            u = jnp.dot(p, vm, preferred_element_type=F32)
            dexp = jnp.dot(p, hm, preferred_element_type=F32)
            rows.append(u / dexp)
        ao = jnp.concatenate(rows, axis=0)
        x = x + jnp.dot(ao, wo, preferred_element_type=F32) + bo
        x = _ln(x, g1, be1)
        y = jnp.dot(x, w1, preferred_element_type=F32) + b1
        y = _gelu(y)
        y = jnp.dot(y, w2, preferred_element_type=F32) + b2
        x = _ln(x + y, g2, be2)
    return _ln(x, ng_ref[...], nb_ref[...])


def _expert_body(Lp1, In1, Lp2, pn, patch, RB, *refs):
    NW = 16 * E_LAYERS
    x1_ref = refs[0]
    wpe1 = refs[1][...]
    wpe2 = refs[2][...]
    pe1 = refs[3][...]
    pe2 = refs[4][...]
    l1refs = refs[5:5 + NW]
    n1g, n1b = refs[5 + NW], refs[6 + NW]
    l2refs = refs[7 + NW:7 + 2 * NW]
    n2g, n2b = refs[7 + 2 * NW], refs[8 + 2 * NW]
    out_ref = refs[-1]

    x1 = x1_ref[...]
    e1 = jnp.dot(x1, wpe1, preferred_element_type=F32)
    e1 = e1 + jnp.concatenate([pe1] * RB, axis=0)
    e2rows = []
    for r in range(RB):
        x1r = x1[r * Lp1:(r + 1) * Lp1, :]
        e2rows.append(jax.lax.dot_general(
            x1r, wpe2, (((0,), (0,)), ((), ())),
            preferred_element_type=F32) + pe2)
    e2 = jnp.concatenate(e2rows, axis=0)

    o1 = _enc_stack(e1, Lp1, pn, RB, l1refs, n1g, n1b)
    o2 = _enc_stack(e2, Lp2, patch, RB, l2refs, n2g, n2b)

    tt = pn + patch
    for r in range(RB):
        t1 = jnp.transpose(o1[r * Lp1:(r + 1) * Lp1, :])
        t2 = jnp.transpose(o2[r * Lp2:(r + 1) * Lp2, :])
        out_ref[r, :, 0:pn] = t1[:, 0:pn]
        out_ref[r, :, pn:tt] = t2[:, 0:patch]


def _pack_layers(layers):
    packed = []
    for p in layers:
        for wname, bname in (("Wq", "bq"), ("Wk", "bk"), ("Wv", "bv"),
                             ("Wo", "bo"), ("W1", "b1"), ("W2", "b2")):
            packed.append(p[wname])
            packed.append(p[bname].reshape(1, -1))
        for g in ("g1", "be1", "g2", "be2"):
            packed.append(p[g].reshape(1, D_MODEL))
    return packed


def _expert_call(x1_2d, wpe1, wpe2, pe1, pe2, l1packed, n1g, n1b,
                 l2packed, n2g, n2b, Lp1, In1, Lp2, pn, patch, RB):
    body = functools.partial(_expert_body, Lp1, In1, Lp2, pn, patch, RB)
    full = lambda a: pl.BlockSpec(a.shape, lambda i: tuple(0 for _ in a.shape))
    args = [x1_2d, wpe1, wpe2, pe1, pe2, *l1packed, n1g, n1b,
            *l2packed, n2g, n2b]
    in_specs = [pl.BlockSpec((RB * Lp1, In1), lambda i: (i, 0))]
    in_specs += [full(a) for a in args[1:]]
    tt = pn + patch
    return pl.pallas_call(
        body,
        grid=(NROWS // RB,),
        in_specs=in_specs,
        out_specs=pl.BlockSpec((RB, D_MODEL, tt), lambda i: (i, 0, 0)),
        out_shape=jax.ShapeDtypeStruct((NROWS, D_MODEL, tt), F32),
    )(*args)


def _head_body(e0, w0, b0, e1, w1, b1, e2, w2, b2, g_ref, r_ref, out_ref):
    kb = pl.program_id(0)
    g56 = jnp.dot(r_ref[...], g_ref[...], preferred_element_type=F32)

    @pl.when(kb == 0)
    def _init():
        out_ref[...] = (g56[:, 0:1] * b0[...] + g56[:, 1:2] * b1[...]
                        + g56[:, 2:3] * b2[...])

    acc = (g56[:, 0:1] * jnp.dot(e0[...], w0[...],
                                 preferred_element_type=F32)
           + g56[:, 1:2] * jnp.dot(e1[...], w1[...],
                                   preferred_element_type=F32)
           + g56[:, 2:3] * jnp.dot(e2[...], w2[...],
                                   preferred_element_type=F32))
    out_ref[...] = out_ref[...] + acc


_HEAD_NB = 2


def _head_call(ecats, whs, bhs, gates, rmat):
    full0 = lambda a: pl.BlockSpec(a.shape,
                                   lambda kb: tuple(0 for _ in a.shape))
    in_specs = []
    args = []
    for e in range(NE):
        kb_e = ecats[e].shape[1] // _HEAD_NB
        in_specs.append(pl.BlockSpec((NROWS, kb_e), lambda kb: (0, kb)))
        in_specs.append(pl.BlockSpec((kb_e, PRED_LEN), lambda kb: (kb, 0)))
        in_specs.append(full0(bhs[e]))
        args += [ecats[e], whs[e], bhs[e]]
    in_specs += [full0(gates), full0(rmat)]
    args += [gates, rmat]
    return pl.pallas_call(
        _head_body,
        grid=(_HEAD_NB,),
        in_specs=in_specs,
        out_specs=pl.BlockSpec((NROWS, PRED_LEN), lambda kb: (0, 0)),
        out_shape=jax.ShapeDtypeStruct((NROWS, PRED_LEN), F32),
    )(*args)


def _pad_to(a, rows, cols):
    return jnp.pad(a, ((0, rows - a.shape[0]), (0, cols - a.shape[1])))


def kernel(x, params):
    xs = x[..., 0].transpose(0, 2, 1).reshape(NROWS, SEQ_LEN)

    cs = jnp.asarray(_CS_NP)
    wg = _pad_to(params["w_gate"], _KP, 128)
    m = jnp.asarray(_M_NP)
    gates_pad, aux_pad = _gate_call(xs, cs, wg, m)
    aux = aux_pad[0, 0]

    xp = jnp.concatenate(
        [xs, jnp.repeat(xs[:, -1:], PAD, axis=1)], axis=1)
    x8 = xp.reshape(NROWS, (SEQ_LEN + PAD) // STRIDE, STRIDE)

    rb = 14
    ecats = []
    whs = []
    bhs = []
    for e, patch in enumerate(PATCHES):
        p = params["experts"][e]
        pn = _pn_of(patch)
        lp1 = _rup(pn, 8)
        lp2 = _rup(patch, 8)
        inp1 = _rup(patch, 8)
        if patch == 16:
            x1u = jnp.concatenate([x8[:, 0:pn, :], x8[:, 1:pn + 1, :]],
                                  axis=-1)
        else:
            x1u = x8[:, 0:pn, :patch]
        x1 = jnp.pad(x1u, ((0, 0), (0, lp1 - pn), (0, inp1 - patch)))

        wpe1 = _pad_to(p["W_pe1"], inp1, D_MODEL)
        wpe2 = _pad_to(p["W_pe2"], lp1, D_MODEL)
        pe1 = jnp.asarray(np.pad(_PE_NP[:pn], ((0, lp1 - pn), (0, 0))))
        pe2 = jnp.asarray(np.pad(_PE_NP[:patch], ((0, lp2 - patch), (0, 0))))

        oc = _expert_call(x1.reshape(NROWS * lp1, inp1), wpe1, wpe2, pe1, pe2,
                          _pack_layers(p["enc1"]),
                          p["n1"]["g"].reshape(1, D_MODEL),
                          p["n1"]["b"].reshape(1, D_MODEL),
                          _pack_layers(p["enc2"]),
                          p["n2"]["g"].reshape(1, D_MODEL),
                          p["n2"]["b"].reshape(1, D_MODEL),
                          lp1, inp1, lp2, pn, patch, rb)
        ecats.append(oc.reshape(NROWS, D_MODEL * (pn + patch)))
        whs.append(p["Wh"])
        bhs.append(p["bh"].reshape(1, PRED_LEN))

    out56 = _head_call(ecats, whs, bhs, gates_pad, jnp.asarray(_R_NP))
    y = out56.reshape(B, ENC_IN, PRED_LEN).transpose(0, 2, 1)
    return y, aux
